# baseline (speedup 1.0000x reference)
import functools

import jax
import jax.numpy as jnp
from jax import lax
from jax.experimental import pallas as pl
from jax.experimental.pallas import tpu as pltpu
from jax.experimental.pallas import tpu_sc as plsc

_BATCH = 4096
_SEQ = 200
_D = 64
_V = 1000000
_NW = 32
_CH = 128
_TW = 8192


def _tc_relayout(tT):
    grid = (_V + _TW - 1) // _TW

    def body(in_ref, out_ref):
        br = in_ref[...].T.reshape(_TW // 2, 2, _D)
        out_ref[...] = jnp.concatenate([br[:, 0, :], br[:, 1, :]], axis=1)

    return pl.pallas_call(
        body,
        grid=(grid,),
        in_specs=[pl.BlockSpec((_D, _TW), lambda i: (0, i))],
        out_specs=pl.BlockSpec((_TW // 2, 2 * _D), lambda i: (i, 0)),
        out_shape=jax.ShapeDtypeStruct((_V // 2, 2 * _D), jnp.float32),
    )(tT).reshape(_V, _D)


def _emb_call(xt, table):
    mesh = plsc.VectorSubcoreMesh(core_axis_name="c", subcore_axis_name="s")

    @functools.partial(
        pl.kernel,
        mesh=mesh,
        out_type=jax.ShapeDtypeStruct((_SEQ, 8, _NW, 8, _CH), jnp.float32),
        scratch_types=[
            pltpu.VMEM((_SEQ, _CH), jnp.int32),
            pltpu.VMEM((2, _CH, _D), jnp.float32),
            pltpu.VMEM((_D, _CH + 1), jnp.float32),
            pltpu.VMEM((_D, _CH + 1), jnp.float32),
            pltpu.SemaphoreType.DMA,
            pltpu.SemaphoreType.DMA,
            pltpu.SemaphoreType.DMA,
            pltpu.SemaphoreType.DMA,
        ],
        compiler_params=pltpu.CompilerParams(
            use_tc_tiling_on_sc=False, needs_layout_passes=False
        ),
    )
    def k(xt_hbm, table_hbm, out_hbm, idx_v, gbuf, tb0, tb1, g0, g1, w0, w1):
        wid = lax.axis_index("s") * 2 + lax.axis_index("c")
        pltpu.sync_copy(xt_hbm.at[:, wid], idx_v)

        ci = lax.iota(jnp.int32, 16)
        cvec = [16 * kk + ci for kk in range(4)]

        pltpu.async_copy(table_hbm.at[idx_v.at[0]], gbuf.at[0], g0)
        pltpu.async_copy(table_hbm.at[idx_v.at[1]], gbuf.at[1], g1)

        def transpose_block(gb, tb):
            def tgroup(tt, carry):
                for dt in range(4):
                    t = tt * 4 + dt
                    tspl = jnp.full((16,), 0, jnp.int32) + t
                    for kk in range(4):
                        vals = gb[t, pl.ds(16 * kk, 16)]
                        plsc.store_scatter(tb, [cvec[kk], tspl], vals)
                return carry

            lax.fori_loop(0, _CH // 4, tgroup, 0)

        def pair(i, carry):
            for p, gsem, wsem in ((0, g0, w0), (1, g1, w1)):
                s = 2 * i + p
                pltpu.make_async_copy(
                    table_hbm.at[idx_v.at[s]], gbuf.at[p], gsem
                ).wait()

                tb = tb0 if p == 0 else tb1

                @pl.when(s >= 2)
                def _():
                    for tr in range(8):
                        pltpu.make_async_copy(
                            tb.at[pl.ds(tr * 8, 8), pl.ds(0, _CH)],
                            out_hbm.at[s, tr, wid],
                            wsem,
                        ).wait()

                transpose_block(gbuf.at[p], tb0 if p == 0 else tb1)

                @pl.when(s + 2 < _SEQ)
                def _():
                    pltpu.async_copy(
                        table_hbm.at[idx_v.at[s + 2]], gbuf.at[p], gsem
                    )

                for tr in range(8):
                    pltpu.async_copy(
                        tb.at[pl.ds(tr * 8, 8), pl.ds(0, _CH)],
                        out_hbm.at[s, tr, wid],
                        wsem,
                    )
            return carry

        lax.fori_loop(0, _SEQ // 2, pair, 0)

        for p, tb, wsem in ((0, tb0, w0), (1, tb1, w1)):
            for tr in range(8):
                pltpu.make_async_copy(
                    tb.at[pl.ds(tr * 8, 8), pl.ds(0, _CH)],
                    out_hbm.at[_SEQ - 2 + p, tr, wid],
                    wsem,
                ).wait()

    return k(xt, table)


def kernel(x, table):
    xt = jnp.transpose(x).reshape(_SEQ, _NW, _CH).astype(jnp.int32)
    trows = _tc_relayout(jnp.transpose(table))
    out5 = _emb_call(xt, trows)
    return lax.reshape(out5, (_BATCH, _SEQ, _D), dimensions=(2, 4, 0, 1, 3))


# speedup vs baseline: 1.1095x; 1.1095x over previous
import functools

import jax
import jax.numpy as jnp
from jax import lax
from jax.experimental import pallas as pl
from jax.experimental.pallas import tpu as pltpu
from jax.experimental.pallas import tpu_sc as plsc

_BATCH = 4096
_SEQ = 200
_D = 64
_V = 1000000
_NW = 32
_CH = 128
_TW = 8192


def _tc_relayout(tT):
    grid = (_V + _TW - 1) // _TW

    def body(in_ref, out_ref):
        b = in_ref[...].T
        out_ref[...] = jnp.concatenate([b, b], axis=1)

    return pl.pallas_call(
        body,
        grid=(grid,),
        in_specs=[pl.BlockSpec((_D, _TW), lambda i: (0, i))],
        out_specs=pl.BlockSpec((_TW, 2 * _D), lambda i: (i, 0)),
        out_shape=jax.ShapeDtypeStruct((_V, 2 * _D), jnp.float32),
    )(tT)


def _emb_call(xt, table):
    mesh = plsc.VectorSubcoreMesh(core_axis_name="c", subcore_axis_name="s")

    @functools.partial(
        pl.kernel,
        mesh=mesh,
        out_type=jax.ShapeDtypeStruct((_SEQ, 8, _NW, 8, _CH), jnp.float32),
        scratch_types=[
            pltpu.VMEM((_SEQ, _CH), jnp.int32),
            pltpu.VMEM((2, _CH, 2 * _D), jnp.float32),
            pltpu.VMEM((_D, _CH + 1), jnp.float32),
            pltpu.VMEM((_D, _CH + 1), jnp.float32),
            pltpu.SemaphoreType.DMA,
            pltpu.SemaphoreType.DMA,
            pltpu.SemaphoreType.DMA,
            pltpu.SemaphoreType.DMA,
        ],
        compiler_params=pltpu.CompilerParams(
            use_tc_tiling_on_sc=False, needs_layout_passes=False
        ),
    )
    def k(xt_hbm, table_hbm, out_hbm, idx_v, gbuf, tb0, tb1, g0, g1, w0, w1):
        wid = lax.axis_index("s") * 2 + lax.axis_index("c")
        pltpu.sync_copy(xt_hbm.at[:, wid], idx_v)

        ci = lax.iota(jnp.int32, 16)
        cvec = [16 * kk + ci for kk in range(4)]

        pltpu.async_copy(table_hbm.at[idx_v.at[0]], gbuf.at[0], g0)
        pltpu.async_copy(table_hbm.at[idx_v.at[1]], gbuf.at[1], g1)

        def transpose_block(gb, tb):
            def tgroup(tt, carry):
                for dt in range(8):
                    t = tt * 8 + dt
                    tspl = jnp.full((16,), 0, jnp.int32) + t
                    for kk in range(4):
                        vals = gb[t, pl.ds(16 * kk, 16)]
                        plsc.store_scatter(tb, [cvec[kk], tspl], vals)
                return carry

            lax.fori_loop(0, _CH // 8, tgroup, 0)

        def pair(i, carry):
            for p, gsem, wsem in ((0, g0, w0), (1, g1, w1)):
                s = 2 * i + p
                pltpu.make_async_copy(
                    table_hbm.at[idx_v.at[s]], gbuf.at[p], gsem
                ).wait()

                tb = tb0 if p == 0 else tb1

                @pl.when(s >= 2)
                def _():
                    for tr in range(8):
                        pltpu.make_async_copy(
                            tb.at[pl.ds(tr * 8, 8), pl.ds(0, _CH)],
                            out_hbm.at[s, tr, wid],
                            wsem,
                        ).wait()

                transpose_block(gbuf.at[p], tb0 if p == 0 else tb1)

                @pl.when(s + 2 < _SEQ)
                def _():
                    pltpu.async_copy(
                        table_hbm.at[idx_v.at[s + 2]], gbuf.at[p], gsem
                    )

                for tr in range(8):
                    pltpu.async_copy(
                        tb.at[pl.ds(tr * 8, 8), pl.ds(0, _CH)],
                        out_hbm.at[s, tr, wid],
                        wsem,
                    )
            return carry

        lax.fori_loop(0, _SEQ // 2, pair, 0)

        for p, tb, wsem in ((0, tb0, w0), (1, tb1, w1)):
            for tr in range(8):
                pltpu.make_async_copy(
                    tb.at[pl.ds(tr * 8, 8), pl.ds(0, _CH)],
                    out_hbm.at[_SEQ - 2 + p, tr, wid],
                    wsem,
                ).wait()

    return k(xt, table)


def kernel(x, table):
    xt = jnp.transpose(x).reshape(_SEQ, _NW, _CH).astype(jnp.int32)
    trows = _tc_relayout(jnp.transpose(table))
    out5 = _emb_call(xt, trows)
    return lax.reshape(out5, (_BATCH, _SEQ, _D), dimensions=(2, 4, 0, 1, 3))
